# baseline (speedup 1.0000x reference)
import functools

import jax
import jax.numpy as jnp
from jax.experimental import pallas as pl
from jax.experimental.pallas import tpu as pltpu

N = 10000
E = 320000
D_IN = 128
H = 4
C = 128
OUT = 128


def _fc_body(x_ref, w_ref, b_ref, o_ref):
    o_ref[...] = jnp.dot(x_ref[...], w_ref[...],
                         preferred_element_type=jnp.float32) + b_ref[...]


def _fc(x, w, b):
    M = x.shape[0]
    BM = 1000
    return pl.pallas_call(
        _fc_body,
        grid=(M // BM,),
        in_specs=[
            pl.BlockSpec((BM, H * C), lambda i: (i, 0)),
            pl.BlockSpec((H * C, OUT), lambda i: (0, 0)),
            pl.BlockSpec((OUT,), lambda i: (0,)),
        ],
        out_specs=pl.BlockSpec((BM, OUT), lambda i: (i, 0)),
        out_shape=jax.ShapeDtypeStruct((M, OUT), jnp.float32),
    )(x, w, b)


def _gat_conv(x, edge_index, W, att_src, att_dst, bias):
    src = edge_index[0]
    dst = edge_index[1]
    xl = (x @ W).reshape(-1, H, C)
    a_src = (xl * att_src).sum(-1)
    a_dst = (xl * att_dst).sum(-1)
    e = a_src[src] + a_dst[dst]
    e = jax.nn.leaky_relu(e, 0.2)
    e_max = jax.ops.segment_max(e, dst, num_segments=N)
    e_max = jnp.where(jnp.isfinite(e_max), e_max, 0.0)
    ex = jnp.exp(e - e_max[dst])
    denom = jax.ops.segment_sum(ex, dst, num_segments=N)
    alpha = ex / (denom[dst] + 1e-16)
    msg = xl[src] * alpha[:, :, None]
    out = jax.ops.segment_sum(msg, dst, num_segments=N)
    return out.reshape(-1, H * C) + bias


def kernel(user_x, item_x, edge_index, W_u, att_src_u, att_dst_u, b_u,
           W_i, att_src_i, att_dst_i, b_i, fc_W, fc_b):
    u = jax.nn.elu(_gat_conv(user_x, edge_index, W_u, att_src_u, att_dst_u, b_u))
    it = jax.nn.elu(_gat_conv(item_x, edge_index, W_i, att_src_i, att_dst_i, b_i))
    x = jnp.concatenate([u, it], axis=0)
    return _fc(x, fc_W, fc_b)


# speedup vs baseline: 21.6697x; 21.6697x over previous
import functools

import jax
import jax.numpy as jnp
from jax import lax
from jax.experimental import pallas as pl
from jax.experimental.pallas import tpu as pltpu
from jax.experimental.pallas import tpu_sc as plsc

N = 10000
NP = 10240
E = 320000
H = 4
C = 128
DIN = 128
OUT = 128
NTILES = 16
RPT = 160
BRS = 32
ERT = E // 128
EPR = RPT * NTILES
EP = EPR * 128
BN = 1024
STRIPE = NP // NTILES

_mesh = plsc.VectorSubcoreMesh(core_axis_name="c", subcore_axis_name="s")
_sc_params = pltpu.CompilerParams(needs_layout_passes=False)


def _proj_body(x_ref, w_ref, att_ref, xl_ref, a_ref):
    x = x_ref[0]
    w = w_ref[0, 0]
    xl = jnp.dot(x, w, preferred_element_type=jnp.float32)
    xl_ref[0, 0] = xl
    a_ref[0, 0] = jnp.dot(xl, att_ref[0, 0], preferred_element_type=jnp.float32)


def _project(x, W4, att2):
    return pl.pallas_call(
        _proj_body,
        grid=(2, NP // BN, H),
        in_specs=[
            pl.BlockSpec((1, BN, DIN), lambda c, i, h: (c, i, 0)),
            pl.BlockSpec((1, 1, DIN, C), lambda c, i, h: (c, h, 0, 0)),
            pl.BlockSpec((1, 1, C, 2), lambda c, i, h: (c, h, 0, 0)),
        ],
        out_specs=[
            pl.BlockSpec((1, 1, BN, C), lambda c, i, h: (c, h, i, 0)),
            pl.BlockSpec((1, 1, BN, 2), lambda c, i, h: (c, h, i, 0)),
        ],
        out_shape=[
            jax.ShapeDtypeStruct((2, H, NP, C), jnp.float32),
            jax.ShapeDtypeStruct((2, H, NP, 2), jnp.float32),
        ],
    )(x, W4, att2)


EPT2 = E // 32
CH2 = 2000


@functools.partial(
    pl.kernel,
    mesh=_mesh,
    out_type=(
        jax.ShapeDtypeStruct((2 * H * E,), jnp.float32),
        jax.ShapeDtypeStruct((2 * H * 32 * NP,), jnp.float32),
    ),
    scratch_types=[
        pltpu.VMEM((H * NP * 2,), jnp.float32),
        pltpu.VMEM((CH2,), jnp.int32),
        pltpu.VMEM((CH2,), jnp.int32),
        pltpu.VMEM((CH2,), jnp.float32),
        pltpu.VMEM((NP,), jnp.float32),
    ],
    compiler_params=_sc_params,
)
def _edge_weights(a_hbm, src_hbm, dst_hbm, ex_hbm, den_hbm,
                  a_tab, sbuf, dbuf, exbuf, dloc):
    wid = lax.axis_index("c") * NTILES + lax.axis_index("s")
    ebase = wid * EPT2
    for cv in range(2):
        pltpu.sync_copy(a_hbm.at[pl.ds(cv * H * NP * 2, H * NP * 2)], a_tab)
        for h in range(H):
            svoff = jnp.full((16,), h * NP * 2, jnp.int32)
            dvoff = jnp.full((16,), h * NP * 2 + 1, jnp.int32)

            def _zero(i, _):
                dloc[pl.ds(i * 16, 16)] = jnp.zeros((16,), jnp.float32)
                return 0

            lax.fori_loop(0, NP // 16, _zero, 0)

            def _chunk(ch, _, cv=cv, h=h, svoff=svoff, dvoff=dvoff):
                off = ebase + ch * CH2
                pltpu.sync_copy(src_hbm.at[pl.ds(off, CH2)], sbuf)
                pltpu.sync_copy(dst_hbm.at[pl.ds(off, CH2)], dbuf)

                def _vec(v, _):
                    sv = sbuf[pl.ds(v * 16, 16)]
                    dv = dbuf[pl.ds(v * 16, 16)]
                    av = plsc.load_gather(a_tab, [sv * 2 + svoff])
                    bv = plsc.load_gather(a_tab, [dv * 2 + dvoff])
                    e = av + bv
                    e = jnp.where(e < 0.0, e * 0.2, e)
                    ev = jnp.exp(e)
                    exbuf[pl.ds(v * 16, 16)] = ev
                    plsc.addupdate_scatter(dloc, [dv], ev)
                    return 0

                lax.fori_loop(0, CH2 // 16, _vec, 0)
                pltpu.sync_copy(
                    exbuf, ex_hbm.at[pl.ds((cv * H + h) * E + off, CH2)])
                return 0

            lax.fori_loop(0, EPT2 // CH2, _chunk, 0)
            pltpu.sync_copy(
                dloc,
                den_hbm.at[pl.ds(((cv * H + h) * 32 + wid) * NP, NP)])


@functools.partial(
    pl.kernel,
    mesh=_mesh,
    out_type=jax.ShapeDtypeStruct((2, H, NP, C), jnp.float32),
    scratch_types=[
        pltpu.VMEM((BRS, 128), jnp.int32),
        pltpu.VMEM((BRS, 128), jnp.int32),
        pltpu.VMEM((BRS, 128), jnp.float32),
        pltpu.VMEM((128, C), jnp.float32),
        pltpu.VMEM((64, C), jnp.float32),
        pltpu.VMEM((STRIPE,), jnp.float32),
        pltpu.VMEM((STRIPE,), jnp.float32),
        pltpu.VMEM_SHARED((NP, C), jnp.float32),
        pltpu.SemaphoreType.DMA,
    ],
    compiler_params=_sc_params,
)
def _message_pass(xl_hbm, src_hbm, dst_hbm, ex_hbm, den_hbm, acc_hbm,
                  sbig, dbig, exbig, rows, zbuf, dtmp, dloc,
                  spacc, sem):
    cid = lax.axis_index("c")
    sid = lax.axis_index("s")
    rowbase = sid * RPT
    stripe0 = sid * STRIPE

    def _zb(i, _):
        for k in range(8):
            zbuf[i, pl.ds(k * 16, 16)] = jnp.zeros((16,), jnp.float32)
        return 0

    lax.fori_loop(0, 64, _zb, 0)

    for cv in range(2):
        for hh in range(2):
            h = cid * 2 + hh
            toff = (cv * H + h) * NP
            tv = jnp.full((16,), 1, jnp.int32) * toff

            def _zs(i, _):
                pltpu.sync_copy(zbuf, spacc.at[pl.ds(stripe0 + i * 64, 64)])
                return 0

            lax.fori_loop(0, STRIPE // 64, _zs, 0)
            plsc.subcore_barrier()

            exrow0 = ((cv * H + h) * EPR) + rowbase

            def _block(b, _, cv=cv, tv=tv, exrow0=exrow0):
                pltpu.sync_copy(src_hbm.at[pl.ds(rowbase + b * BRS, BRS)],
                                sbig)
                pltpu.sync_copy(dst_hbm.at[pl.ds(rowbase + b * BRS, BRS)],
                                dbig)
                pltpu.sync_copy(ex_hbm.at[pl.ds(exrow0 + b * BRS, BRS)],
                                exbig)

                def _addoff(i, _, tv=tv):
                    for k in range(8):
                        sl = pl.ds(k * 16, 16)
                        sbig[i, sl] = sbig[i, sl] + tv
                    return 0

                lax.fori_loop(0, BRS, _addoff, 0)

                def _chunk(i, _):
                    pltpu.async_copy(xl_hbm.at[sbig.at[i]], rows, sem).wait()

                    def _scale(g, _):
                        exv = exbig[i, pl.ds(g * 16, 16)]
                        for t in range(16):
                            v = jnp.full((16,), exv[t], jnp.float32)
                            j = g * 16 + t
                            for k in range(8):
                                sl = pl.ds(k * 16, 16)
                                rows[j, sl] = rows[j, sl] * v
                        return 0

                    lax.fori_loop(0, 8, _scale, 0)
                    pltpu.sync_copy(rows, spacc.at[dbig.at[i]], add=True)
                    return 0

                lax.fori_loop(0, BRS, _chunk, 0)
                return 0

            lax.fori_loop(0, RPT // BRS, _block, 0)
            plsc.subcore_barrier()

            dbase = (cv * H + h) * 32

            def _zd(i, _):
                dloc[pl.ds(i * 16, 16)] = jnp.zeros((16,), jnp.float32)
                return 0

            lax.fori_loop(0, STRIPE // 16, _zd, 0)
            for r in range(32):
                pltpu.sync_copy(
                    den_hbm.at[pl.ds((dbase + r) * NP + stripe0, STRIPE)],
                    dtmp)

                def _dred(i, _):
                    sl = pl.ds(i * 16, 16)
                    dloc[sl] = dloc[sl] + dtmp[sl]
                    return 0

                lax.fori_loop(0, STRIPE // 16, _dred, 0)

            def _flush(i, _, cv=cv, h=h):
                r0 = stripe0 + i * 64
                frows = rows.at[pl.ds(0, 64)]
                pltpu.sync_copy(spacc.at[pl.ds(r0, 64)], frows)

                def _norm(g, _):
                    dvec = dloc[pl.ds(i * 64 + g * 16, 16)]
                    ivec = 1.0 / (dvec + 1e-16)
                    for t in range(16):
                        iv = jnp.full((16,), ivec[t], jnp.float32)
                        j = g * 16 + t
                        for k in range(8):
                            sl = pl.ds(k * 16, 16)
                            rows[j, sl] = rows[j, sl] * iv
                    return 0

                lax.fori_loop(0, 4, _norm, 0)
                pltpu.sync_copy(frows, acc_hbm.at[cv, h, pl.ds(r0, 64)])
                return 0

            lax.fori_loop(0, STRIPE // 64, _flush, 0)
            plsc.subcore_barrier()


def _fc_body(acc_ref, bias_ref, fcw_ref, fcb_ref, o_ref):
    h = pl.program_id(2)
    z = acc_ref[0, 0] + bias_ref[0, 0]
    z = jnp.where(z > 0.0, z, jnp.exp(jnp.minimum(z, 0.0)) - 1.0)
    part = jnp.dot(z, fcw_ref[0], preferred_element_type=jnp.float32)

    @pl.when(h == 0)
    def _():
        o_ref[0] = part + fcb_ref[...][None, :]

    @pl.when(h > 0)
    def _():
        o_ref[0] = o_ref[0] + part


def _fc(accn, bias3, fcw3, fcb):
    return pl.pallas_call(
        _fc_body,
        grid=(2, NP // BN, H),
        in_specs=[
            pl.BlockSpec((1, 1, BN, C), lambda c, i, h: (c, h, i, 0)),
            pl.BlockSpec((1, 1, 1, C), lambda c, i, h: (c, h, 0, 0)),
            pl.BlockSpec((1, C, OUT), lambda c, i, h: (h, 0, 0)),
            pl.BlockSpec((OUT,), lambda c, i, h: (0,)),
        ],
        out_specs=pl.BlockSpec((1, BN, OUT), lambda c, i, h: (c, i, 0)),
        out_shape=jax.ShapeDtypeStruct((2, NP, OUT), jnp.float32),
    )(accn, bias3, fcw3, fcb)


def kernel(user_x, item_x, edge_index, W_u, att_src_u, att_dst_u, b_u,
           W_i, att_src_i, att_dst_i, b_i, fc_W, fc_b):
    x = jnp.pad(jnp.stack([user_x, item_x]), ((0, 0), (0, NP - N), (0, 0)))
    W4 = jnp.stack([W_u, W_i]).reshape(2, DIN, H, C).transpose(0, 2, 1, 3)
    att2 = jnp.stack([
        jnp.stack([att_src_u, att_dst_u], axis=-1),
        jnp.stack([att_src_i, att_dst_i], axis=-1),
    ])
    xl, a = _project(x, W4, att2)

    src = edge_index[0]
    dst = edge_index[1]
    ex, den = _edge_weights(a.reshape(-1), src, dst)

    srcp = jnp.pad(src, (0, EP - E)).reshape(EPR, 128)
    dstp = jnp.pad(dst, (0, EP - E)).reshape(EPR, 128)
    ex3 = jnp.pad(ex.reshape(2 * H, ERT, 128),
                  ((0, 0), (0, RPT * NTILES - ERT), (0, 0))
                  ).reshape(2 * H * EPR, 128)
    accn = _message_pass(xl.reshape(2 * H * NP, C), srcp, dstp, ex3, den)

    bias3 = jnp.stack([b_u, b_i]).reshape(2, H, 1, C)
    outp = _fc(accn, bias3, fc_W.reshape(H, C, OUT), fc_b)
    return outp[:, :N, :].reshape(2 * N, OUT)


# speedup vs baseline: 25.7045x; 1.1862x over previous
import functools

import jax
import jax.numpy as jnp
from jax import lax
from jax.experimental import pallas as pl
from jax.experimental.pallas import tpu as pltpu
from jax.experimental.pallas import tpu_sc as plsc

N = 10000
NP = 10240
E = 320000
H = 4
C = 128
DIN = 128
OUT = 128
NTILES = 16
RPT = 160
BRS = 32
ERT = E // 128
EPR = RPT * NTILES
EP = EPR * 128
BN = 1024
STRIPE = NP // NTILES

_mesh = plsc.VectorSubcoreMesh(core_axis_name="c", subcore_axis_name="s")
_sc_params = pltpu.CompilerParams(needs_layout_passes=False)


def _proj_body(x_ref, w_ref, att_ref, xl_ref, a_ref):
    x = x_ref[0]
    w = w_ref[0, 0]
    xl = jnp.dot(x, w, preferred_element_type=jnp.float32)
    xl_ref[0, 0] = xl
    a_ref[0, 0] = jnp.dot(xl, att_ref[0, 0], preferred_element_type=jnp.float32)


def _project(x, W4, att2):
    return pl.pallas_call(
        _proj_body,
        grid=(2, NP // BN, H),
        in_specs=[
            pl.BlockSpec((1, BN, DIN), lambda c, i, h: (c, i, 0)),
            pl.BlockSpec((1, 1, DIN, C), lambda c, i, h: (c, h, 0, 0)),
            pl.BlockSpec((1, 1, C, 2), lambda c, i, h: (c, h, 0, 0)),
        ],
        out_specs=[
            pl.BlockSpec((1, 1, BN, C), lambda c, i, h: (c, h, i, 0)),
            pl.BlockSpec((1, 1, BN, 2), lambda c, i, h: (c, h, i, 0)),
        ],
        out_shape=[
            jax.ShapeDtypeStruct((2, H, NP, C), jnp.float32),
            jax.ShapeDtypeStruct((2, H, NP, 2), jnp.float32),
        ],
    )(x, W4, att2)


EPT2 = E // 32
CH2 = 2000


@functools.partial(
    pl.kernel,
    mesh=_mesh,
    out_type=(
        jax.ShapeDtypeStruct((2 * H * E,), jnp.float32),
        jax.ShapeDtypeStruct((2 * H * 32 * NP,), jnp.float32),
    ),
    scratch_types=[
        pltpu.VMEM((H * NP * 2,), jnp.float32),
        pltpu.VMEM((CH2,), jnp.int32),
        pltpu.VMEM((CH2,), jnp.int32),
        pltpu.VMEM((CH2,), jnp.float32),
        pltpu.VMEM((NP,), jnp.float32),
    ],
    compiler_params=_sc_params,
)
def _edge_weights(a_hbm, src_hbm, dst_hbm, ex_hbm, den_hbm,
                  a_tab, sbuf, dbuf, exbuf, dloc):
    wid = lax.axis_index("c") * NTILES + lax.axis_index("s")
    ebase = wid * EPT2
    for cv in range(2):
        pltpu.sync_copy(a_hbm.at[pl.ds(cv * H * NP * 2, H * NP * 2)], a_tab)
        for h in range(H):
            svoff = jnp.full((16,), h * NP * 2, jnp.int32)
            dvoff = jnp.full((16,), h * NP * 2 + 1, jnp.int32)

            def _zero(i, _):
                dloc[pl.ds(i * 16, 16)] = jnp.zeros((16,), jnp.float32)
                return 0

            lax.fori_loop(0, NP // 16, _zero, 0)

            def _chunk(ch, _, cv=cv, h=h, svoff=svoff, dvoff=dvoff):
                off = ebase + ch * CH2
                pltpu.sync_copy(src_hbm.at[pl.ds(off, CH2)], sbuf)
                pltpu.sync_copy(dst_hbm.at[pl.ds(off, CH2)], dbuf)

                def _vec(v, _):
                    sv = sbuf[pl.ds(v * 16, 16)]
                    dv = dbuf[pl.ds(v * 16, 16)]
                    av = plsc.load_gather(a_tab, [sv * 2 + svoff])
                    bv = plsc.load_gather(a_tab, [dv * 2 + dvoff])
                    e = av + bv
                    e = jnp.where(e < 0.0, e * 0.2, e)
                    ev = jnp.exp(e)
                    exbuf[pl.ds(v * 16, 16)] = ev
                    plsc.addupdate_scatter(dloc, [dv], ev)
                    return 0

                lax.fori_loop(0, CH2 // 16, _vec, 0)
                pltpu.sync_copy(
                    exbuf, ex_hbm.at[pl.ds((cv * H + h) * E + off, CH2)])
                return 0

            lax.fori_loop(0, EPT2 // CH2, _chunk, 0)
            pltpu.sync_copy(
                dloc,
                den_hbm.at[pl.ds(((cv * H + h) * 32 + wid) * NP, NP)])


@functools.partial(
    pl.kernel,
    mesh=_mesh,
    out_type=jax.ShapeDtypeStruct((2, H, NP, C), jnp.float32),
    scratch_types=[
        pltpu.VMEM((BRS, 128), jnp.int32),
        pltpu.VMEM((BRS, 128), jnp.int32),
        pltpu.VMEM((BRS, 128), jnp.float32),
        pltpu.VMEM((128, C), jnp.float32),
        pltpu.VMEM((128, C), jnp.float32),
        pltpu.VMEM((8, C), jnp.float32),
        pltpu.VMEM((STRIPE,), jnp.float32),
        pltpu.VMEM((STRIPE,), jnp.float32),
        pltpu.VMEM_SHARED((NP, C), jnp.float32),
        pltpu.SemaphoreType.DMA,
        pltpu.SemaphoreType.DMA,
        pltpu.SemaphoreType.DMA,
        pltpu.SemaphoreType.DMA,
    ],
    compiler_params=_sc_params,
)
def _message_pass(xl_hbm, src_hbm, dst_hbm, ex_hbm, den_hbm, acc_hbm,
                  sbig, dbig, exbig, rows, rowsb, zbuf, dtmp, dloc,
                  spacc, semga, semgb, semsa, semsb):
    cid = lax.axis_index("c")
    sid = lax.axis_index("s")
    rowbase = sid * RPT
    stripe0 = sid * STRIPE

    def _zb(i, _):
        for k in range(8):
            zbuf[i, pl.ds(k * 16, 16)] = jnp.zeros((16,), jnp.float32)
        return 0

    lax.fori_loop(0, 8, _zb, 0)

    def _gissue(kchunk, rbuf, gsem):
        pltpu.async_copy(xl_hbm.at[sbig.at[kchunk]], rbuf, gsem)

    def _gwait(rbuf, gsem):
        pltpu.make_async_copy(xl_hbm.at[pl.ds(0, 128)], rbuf, gsem).wait()

    def _do(kchunk, rbuf, gsem, ssem):
        _gwait(rbuf, gsem)

        def _scale(g, _):
            exv = exbig[kchunk, pl.ds(g * 16, 16)]
            for t in range(16):
                v = jnp.full((16,), exv[t], jnp.float32)
                j = g * 16 + t
                for k in range(8):
                    sl = pl.ds(k * 16, 16)
                    rbuf[j, sl] = rbuf[j, sl] * v
            return 0

        lax.fori_loop(0, 8, _scale, 0)
        pltpu.async_copy(rbuf, spacc.at[dbig.at[kchunk]], ssem, add=True)

    for cv in range(2):
        for hh in range(2):
            h = cid * 2 + hh
            toff = (cv * H + h) * NP
            tv = jnp.full((16,), 1, jnp.int32) * toff

            def _zs(i, _):
                pltpu.sync_copy(zbuf, spacc.at[pl.ds(stripe0 + i * 8, 8)])
                return 0

            lax.fori_loop(0, STRIPE // 8, _zs, 0)
            plsc.subcore_barrier()

            exrow0 = ((cv * H + h) * EPR) + rowbase

            def _block(b, _, cv=cv, tv=tv, exrow0=exrow0):
                pltpu.sync_copy(src_hbm.at[pl.ds(rowbase + b * BRS, BRS)],
                                sbig)
                pltpu.sync_copy(dst_hbm.at[pl.ds(rowbase + b * BRS, BRS)],
                                dbig)
                pltpu.sync_copy(ex_hbm.at[pl.ds(exrow0 + b * BRS, BRS)],
                                exbig)

                def _addoff(i, _, tv=tv):
                    for k in range(8):
                        sl = pl.ds(k * 16, 16)
                        sbig[i, sl] = sbig[i, sl] + tv
                    return 0

                lax.fori_loop(0, BRS, _addoff, 0)

                _gissue(0, rows, semga)
                _gissue(1, rowsb, semgb)

                def _pair(p, _):
                    _do(2 * p, rows, semga, semsa)
                    _do(2 * p + 1, rowsb, semgb, semsb)
                    _gwait(rows, semsa)
                    _gissue(2 * p + 2, rows, semga)
                    _gwait(rowsb, semsb)
                    _gissue(2 * p + 3, rowsb, semgb)
                    return 0

                lax.fori_loop(0, BRS // 2 - 1, _pair, 0)
                _do(BRS - 2, rows, semga, semsa)
                _do(BRS - 1, rowsb, semgb, semsb)
                _gwait(rows, semsa)
                _gwait(rowsb, semsb)
                return 0

            lax.fori_loop(0, RPT // BRS, _block, 0)
            plsc.subcore_barrier()

            dbase = (cv * H + h) * 32

            def _zd(i, _):
                dloc[pl.ds(i * 16, 16)] = jnp.zeros((16,), jnp.float32)
                return 0

            lax.fori_loop(0, STRIPE // 16, _zd, 0)
            for r in range(32):
                pltpu.sync_copy(
                    den_hbm.at[pl.ds((dbase + r) * NP + stripe0, STRIPE)],
                    dtmp)

                def _dred(i, _):
                    sl = pl.ds(i * 16, 16)
                    dloc[sl] = dloc[sl] + dtmp[sl]
                    return 0

                lax.fori_loop(0, STRIPE // 16, _dred, 0)

            def _flush(i, _, cv=cv, h=h):
                r0 = stripe0 + i * 64
                frows = rows.at[pl.ds(0, 64)]
                pltpu.sync_copy(spacc.at[pl.ds(r0, 64)], frows)

                def _norm(g, _):
                    dvec = dloc[pl.ds(i * 64 + g * 16, 16)]
                    ivec = 1.0 / (dvec + 1e-16)
                    for t in range(16):
                        iv = jnp.full((16,), ivec[t], jnp.float32)
                        j = g * 16 + t
                        for k in range(8):
                            sl = pl.ds(k * 16, 16)
                            rows[j, sl] = rows[j, sl] * iv
                    return 0

                lax.fori_loop(0, 4, _norm, 0)
                pltpu.sync_copy(frows, acc_hbm.at[cv, h, pl.ds(r0, 64)])
                return 0

            lax.fori_loop(0, STRIPE // 64, _flush, 0)
            plsc.subcore_barrier()


def _fc_body(acc_ref, bias_ref, fcw_ref, fcb_ref, o_ref):
    h = pl.program_id(2)
    z = acc_ref[0, 0] + bias_ref[0, 0]
    z = jnp.where(z > 0.0, z, jnp.exp(jnp.minimum(z, 0.0)) - 1.0)
    part = jnp.dot(z, fcw_ref[0], preferred_element_type=jnp.float32)

    @pl.when(h == 0)
    def _():
        o_ref[0] = part + fcb_ref[...][None, :]

    @pl.when(h > 0)
    def _():
        o_ref[0] = o_ref[0] + part


def _fc(accn, bias3, fcw3, fcb):
    return pl.pallas_call(
        _fc_body,
        grid=(2, NP // BN, H),
        in_specs=[
            pl.BlockSpec((1, 1, BN, C), lambda c, i, h: (c, h, i, 0)),
            pl.BlockSpec((1, 1, 1, C), lambda c, i, h: (c, h, 0, 0)),
            pl.BlockSpec((1, C, OUT), lambda c, i, h: (h, 0, 0)),
            pl.BlockSpec((OUT,), lambda c, i, h: (0,)),
        ],
        out_specs=pl.BlockSpec((1, BN, OUT), lambda c, i, h: (c, i, 0)),
        out_shape=jax.ShapeDtypeStruct((2, NP, OUT), jnp.float32),
    )(accn, bias3, fcw3, fcb)


def kernel(user_x, item_x, edge_index, W_u, att_src_u, att_dst_u, b_u,
           W_i, att_src_i, att_dst_i, b_i, fc_W, fc_b):
    x = jnp.pad(jnp.stack([user_x, item_x]), ((0, 0), (0, NP - N), (0, 0)))
    W4 = jnp.stack([W_u, W_i]).reshape(2, DIN, H, C).transpose(0, 2, 1, 3)
    att2 = jnp.stack([
        jnp.stack([att_src_u, att_dst_u], axis=-1),
        jnp.stack([att_src_i, att_dst_i], axis=-1),
    ])
    xl, a = _project(x, W4, att2)

    src = edge_index[0]
    dst = edge_index[1]
    ex, den = _edge_weights(a.reshape(-1), src, dst)

    srcp = jnp.pad(src, (0, EP - E)).reshape(EPR, 128)
    dstp = jnp.pad(dst, (0, EP - E)).reshape(EPR, 128)
    ex3 = jnp.pad(ex.reshape(2 * H, ERT, 128),
                  ((0, 0), (0, RPT * NTILES - ERT), (0, 0))
                  ).reshape(2 * H * EPR, 128)
    accn = _message_pass(xl.reshape(2 * H * NP, C), srcp, dstp, ex3, den)

    bias3 = jnp.stack([b_u, b_i]).reshape(2, H, 1, C)
    outp = _fc(accn, bias3, fc_W.reshape(H, C, OUT), fc_b)
    return outp[:, :N, :].reshape(2 * N, OUT)


# speedup vs baseline: 26.5636x; 1.0334x over previous
import functools

import jax
import jax.numpy as jnp
from jax import lax
from jax.experimental import pallas as pl
from jax.experimental.pallas import tpu as pltpu
from jax.experimental.pallas import tpu_sc as plsc

N = 10000
NP = 10240
E = 320000
H = 4
C = 128
DIN = 128
OUT = 128
NTILES = 16
RPT = 160
BRS = 32
ERT = E // 128
EPR = RPT * NTILES
EP = EPR * 128
BN = 1024
STRIPE = NP // NTILES

_mesh = plsc.VectorSubcoreMesh(core_axis_name="c", subcore_axis_name="s")
_sc_params = pltpu.CompilerParams(needs_layout_passes=False)


def _proj_body(x_ref, w_ref, att_ref, xl_ref, a_ref):
    x = x_ref[0]
    w = w_ref[0, 0]
    xl = jnp.dot(x, w, preferred_element_type=jnp.float32)
    xl_ref[0, 0] = xl
    a_ref[0, 0] = jnp.dot(xl, att_ref[0, 0], preferred_element_type=jnp.float32)


def _project(x, W4, att2):
    return pl.pallas_call(
        _proj_body,
        grid=(2, NP // BN, H),
        in_specs=[
            pl.BlockSpec((1, BN, DIN), lambda c, i, h: (c, i, 0)),
            pl.BlockSpec((1, 1, DIN, C), lambda c, i, h: (c, h, 0, 0)),
            pl.BlockSpec((1, 1, C, 2), lambda c, i, h: (c, h, 0, 0)),
        ],
        out_specs=[
            pl.BlockSpec((1, 1, BN, C), lambda c, i, h: (c, h, i, 0)),
            pl.BlockSpec((1, 1, BN, 2), lambda c, i, h: (c, h, i, 0)),
        ],
        out_shape=[
            jax.ShapeDtypeStruct((2, H, NP, C), jnp.float32),
            jax.ShapeDtypeStruct((2, H, NP, 2), jnp.float32),
        ],
    )(x, W4, att2)


EPT2 = E // 32
CH2 = 2000


@functools.partial(
    pl.kernel,
    mesh=_mesh,
    out_type=(
        jax.ShapeDtypeStruct((2 * H * E,), jnp.float32),
        jax.ShapeDtypeStruct((2 * H * 32 * NP,), jnp.float32),
    ),
    scratch_types=[
        pltpu.VMEM((H * NP * 2,), jnp.float32),
        pltpu.VMEM((CH2,), jnp.int32),
        pltpu.VMEM((CH2,), jnp.int32),
        pltpu.VMEM((CH2,), jnp.float32),
        pltpu.VMEM((NP,), jnp.float32),
    ],
    compiler_params=_sc_params,
)
def _edge_weights(a_hbm, src_hbm, dst_hbm, ex_hbm, den_hbm,
                  a_tab, sbuf, dbuf, exbuf, dloc):
    wid = lax.axis_index("c") * NTILES + lax.axis_index("s")
    ebase = wid * EPT2
    for cv in range(2):
        pltpu.sync_copy(a_hbm.at[pl.ds(cv * H * NP * 2, H * NP * 2)], a_tab)
        for h in range(H):
            svoff = jnp.full((16,), h * NP * 2, jnp.int32)
            dvoff = jnp.full((16,), h * NP * 2 + 1, jnp.int32)

            def _zero(i, _):
                dloc[pl.ds(i * 16, 16)] = jnp.zeros((16,), jnp.float32)
                return 0

            lax.fori_loop(0, NP // 16, _zero, 0)

            def _chunk(ch, _, cv=cv, h=h, svoff=svoff, dvoff=dvoff):
                off = ebase + ch * CH2
                pltpu.sync_copy(src_hbm.at[pl.ds(off, CH2)], sbuf)
                pltpu.sync_copy(dst_hbm.at[pl.ds(off, CH2)], dbuf)

                def _vec(v, _):
                    sv = sbuf[pl.ds(v * 16, 16)]
                    dv = dbuf[pl.ds(v * 16, 16)]
                    av = plsc.load_gather(a_tab, [sv * 2 + svoff])
                    bv = plsc.load_gather(a_tab, [dv * 2 + dvoff])
                    e = av + bv
                    e = jnp.where(e < 0.0, e * 0.2, e)
                    ev = jnp.exp(e)
                    exbuf[pl.ds(v * 16, 16)] = ev
                    plsc.addupdate_scatter(dloc, [dv], ev)
                    return 0

                lax.fori_loop(0, CH2 // 16, _vec, 0)
                pltpu.sync_copy(
                    exbuf, ex_hbm.at[pl.ds((cv * H + h) * E + off, CH2)])
                return 0

            lax.fori_loop(0, EPT2 // CH2, _chunk, 0)
            pltpu.sync_copy(
                dloc,
                den_hbm.at[pl.ds(((cv * H + h) * 32 + wid) * NP, NP)])


@functools.partial(
    pl.kernel,
    mesh=_mesh,
    out_type=jax.ShapeDtypeStruct((2, H, NP, C), jnp.float32),
    scratch_types=[
        pltpu.VMEM((BRS, 128), jnp.int32),
        pltpu.VMEM((BRS, 128), jnp.int32),
        pltpu.VMEM((BRS, 128), jnp.float32),
        pltpu.VMEM((128, C), jnp.float32),
        pltpu.VMEM((128, C), jnp.float32),
        pltpu.VMEM((STRIPE,), jnp.float32),
        pltpu.VMEM_SHARED((NP, C), jnp.float32),
        pltpu.SemaphoreType.DMA,
        pltpu.SemaphoreType.DMA,
        pltpu.SemaphoreType.DMA,
        pltpu.SemaphoreType.DMA,
    ],
    compiler_params=_sc_params,
)
def _message_pass(xl_hbm, src_hbm, dst_hbm, ex_hbm, den_hbm, acc_hbm,
                  sbig, dbig, exbig, rows, rowsb, dloc,
                  spacc, semga, semgb, semsa, semsb):
    cid = lax.axis_index("c")
    sid = lax.axis_index("s")
    rowbase = sid * RPT
    stripe0 = sid * STRIPE

    def _gissue(kchunk, rbuf, gsem):
        pltpu.async_copy(xl_hbm.at[sbig.at[kchunk]], rbuf, gsem)

    def _gwait(rbuf, gsem):
        pltpu.make_async_copy(xl_hbm.at[pl.ds(0, 128)], rbuf, gsem).wait()

    def _do(kchunk, rbuf, gsem):
        _gwait(rbuf, gsem)

        def _scale(g, _):
            exv = exbig[kchunk, pl.ds(g * 16, 16)]
            for t in range(16):
                v = jnp.full((16,), exv[t], jnp.float32)
                j = g * 16 + t
                for k in range(8):
                    sl = pl.ds(k * 16, 16)
                    rbuf[j, sl] = rbuf[j, sl] * v
            return 0

        lax.fori_loop(0, 8, _scale, 0)

    for cv in range(2):
        for hh in range(2):
            h = cid * 2 + hh
            toff = (cv * H + h) * NP
            tv = jnp.full((16,), 1, jnp.int32) * toff

            def _zr(i, _):
                for k in range(8):
                    rows[i, pl.ds(k * 16, 16)] = jnp.zeros((16,), jnp.float32)
                return 0

            lax.fori_loop(0, 128, _zr, 0)

            def _zs(i, _):
                pltpu.sync_copy(rows, spacc.at[pl.ds(stripe0 + i * 128, 128)])
                return 0

            lax.fori_loop(0, STRIPE // 128, _zs, 0)
            plsc.subcore_barrier()

            exrow0 = ((cv * H + h) * EPR) + rowbase

            def _block(b, _, cv=cv, tv=tv, exrow0=exrow0):
                pltpu.sync_copy(src_hbm.at[pl.ds(rowbase + b * BRS, BRS)],
                                sbig)
                pltpu.sync_copy(dst_hbm.at[pl.ds(rowbase + b * BRS, BRS)],
                                dbig)
                pltpu.sync_copy(ex_hbm.at[pl.ds(exrow0 + b * BRS, BRS)],
                                exbig)

                def _addoff(i, _, tv=tv):
                    for k in range(8):
                        sl = pl.ds(k * 16, 16)
                        sbig[i, sl] = sbig[i, sl] + tv
                    return 0

                lax.fori_loop(0, BRS, _addoff, 0)

                _gissue(0, rows, semga)
                _gissue(1, rowsb, semgb)

                def _pair(p, _):
                    _do(2 * p, rows, semga)
                    pltpu.async_copy(rows, spacc.at[dbig.at[2 * p]],
                                     semsa, add=True)
                    _do(2 * p + 1, rowsb, semgb)
                    _gwait(rows, semsa)
                    _gissue(2 * p + 2, rows, semga)
                    pltpu.async_copy(rowsb, spacc.at[dbig.at[2 * p + 1]],
                                     semsb, add=True)
                    _gwait(rowsb, semsb)
                    _gissue(2 * p + 3, rowsb, semgb)
                    return 0

                lax.fori_loop(0, BRS // 2 - 1, _pair, 0)
                _do(BRS - 2, rows, semga)
                pltpu.async_copy(rows, spacc.at[dbig.at[BRS - 2]],
                                 semsa, add=True)
                _do(BRS - 1, rowsb, semgb)
                _gwait(rows, semsa)
                pltpu.async_copy(rowsb, spacc.at[dbig.at[BRS - 1]],
                                 semsb, add=True)
                _gwait(rowsb, semsb)
                return 0

            lax.fori_loop(0, RPT // BRS, _block, 0)
            plsc.subcore_barrier()

            pltpu.sync_copy(
                den_hbm.at[pl.ds((cv * H + h) * NP + stripe0, STRIPE)],
                dloc)

            def _flush(i, _, cv=cv, h=h):
                r0 = stripe0 + i * 128
                pltpu.sync_copy(spacc.at[pl.ds(r0, 128)], rows)

                def _norm(g, _):
                    ivec = dloc[pl.ds(i * 128 + g * 16, 16)]
                    for t in range(16):
                        iv = jnp.full((16,), ivec[t], jnp.float32)
                        j = g * 16 + t
                        for k in range(8):
                            sl = pl.ds(k * 16, 16)
                            rows[j, sl] = rows[j, sl] * iv
                    return 0

                lax.fori_loop(0, 8, _norm, 0)
                pltpu.sync_copy(rows, acc_hbm.at[cv, h, pl.ds(r0, 128)])
                return 0

            lax.fori_loop(0, STRIPE // 128, _flush, 0)
            plsc.subcore_barrier()


def _denr_body(d_ref, o_ref):
    s = jnp.sum(d_ref[...], axis=0)
    o_ref[...] = 1.0 / (s + 1e-16)


def _den_reduce(den1):
    return pl.pallas_call(
        _denr_body,
        grid=(2 * H, NP // BN),
        in_specs=[pl.BlockSpec((32, BN), lambda j, i: (j, i))],
        out_specs=pl.BlockSpec((BN,), lambda j, i: (j * (NP // BN) + i,)),
        out_shape=jax.ShapeDtypeStruct((2 * H * NP,), jnp.float32),
    )(den1)


def _fc_body(acc_ref, bias_ref, fcw_ref, fcb_ref, o_ref):
    h = pl.program_id(2)
    z = acc_ref[0, 0] + bias_ref[0, 0]
    z = jnp.where(z > 0.0, z, jnp.exp(jnp.minimum(z, 0.0)) - 1.0)
    part = jnp.dot(z, fcw_ref[0], preferred_element_type=jnp.float32)

    @pl.when(h == 0)
    def _():
        o_ref[0] = part + fcb_ref[...][None, :]

    @pl.when(h > 0)
    def _():
        o_ref[0] = o_ref[0] + part


def _fc(accn, bias3, fcw3, fcb):
    return pl.pallas_call(
        _fc_body,
        grid=(2, NP // BN, H),
        in_specs=[
            pl.BlockSpec((1, 1, BN, C), lambda c, i, h: (c, h, i, 0)),
            pl.BlockSpec((1, 1, 1, C), lambda c, i, h: (c, h, 0, 0)),
            pl.BlockSpec((1, C, OUT), lambda c, i, h: (h, 0, 0)),
            pl.BlockSpec((OUT,), lambda c, i, h: (0,)),
        ],
        out_specs=pl.BlockSpec((1, BN, OUT), lambda c, i, h: (c, i, 0)),
        out_shape=jax.ShapeDtypeStruct((2, NP, OUT), jnp.float32),
    )(accn, bias3, fcw3, fcb)


def kernel(user_x, item_x, edge_index, W_u, att_src_u, att_dst_u, b_u,
           W_i, att_src_i, att_dst_i, b_i, fc_W, fc_b):
    x = jnp.pad(jnp.stack([user_x, item_x]), ((0, 0), (0, NP - N), (0, 0)))
    W4 = jnp.stack([W_u, W_i]).reshape(2, DIN, H, C).transpose(0, 2, 1, 3)
    att2 = jnp.stack([
        jnp.stack([att_src_u, att_dst_u], axis=-1),
        jnp.stack([att_src_i, att_dst_i], axis=-1),
    ])
    xl, a = _project(x, W4, att2)

    src = edge_index[0]
    dst = edge_index[1]
    ex, den = _edge_weights(a.reshape(-1), src, dst)

    srcp = jnp.pad(src, (0, EP - E)).reshape(EPR, 128)
    dstp = jnp.pad(dst, (0, EP - E)).reshape(EPR, 128)
    ex3 = jnp.pad(ex.reshape(2 * H, ERT, 128),
                  ((0, 0), (0, RPT * NTILES - ERT), (0, 0))
                  ).reshape(2 * H * EPR, 128)
    den2 = _den_reduce(den.reshape(2 * H * 32, NP))
    accn = _message_pass(xl.reshape(2 * H * NP, C), srcp, dstp, ex3, den2)

    bias3 = jnp.stack([b_u, b_i]).reshape(2, H, 1, C)
    outp = _fc(accn, bias3, fc_W.reshape(H, C, OUT), fc_b)
    return outp[:, :N, :].reshape(2 * N, OUT)
